# baseline (speedup 1.0000x reference)
import functools

import jax
import jax.numpy as jnp
from jax import lax
from jax.experimental import pallas as pl
from jax.experimental.pallas import tpu as pltpu
from jax.experimental.pallas import tpu_sc as plsc

_HID = 64
_L = 200
_B = 4096
_NW = 32
_ROWS_PER_W = _B // _NW
_NBUF = 4
_N0, _N1 = 104, 96


def _tpe_body(x_hbm, tok_hbm, pos_hbm, out_hbm, *scratch):
  bufs = scratch[0:_NBUF]
  idxa = scratch[_NBUF:2 * _NBUF]
  idxb = scratch[2 * _NBUF:3 * _NBUF]
  pos_v = scratch[3 * _NBUF]
  gsems = scratch[3 * _NBUF + 1:3 * _NBUF + 1 + _NBUF]
  ssems = scratch[3 * _NBUF + 1 + _NBUF:]

  wid = lax.axis_index("s") * 2 + lax.axis_index("c")
  row0 = wid * _ROWS_PER_W

  pltpu.sync_copy(pos_hbm.at[pl.ds(0, _L)], pos_v)

  def start_fetch(g, b):
    base = (row0 + g) * _L
    pltpu.sync_copy(x_hbm.at[pl.ds(base, _N0)], idxa[b])
    pltpu.sync_copy(x_hbm.at[pl.ds(base + _N0, _N1)], idxb[b])
    pltpu.make_async_copy(
        tok_hbm.at[idxa[b]], bufs[b].at[pl.ds(0, _N0)], gsems[b]).start()
    pltpu.make_async_copy(
        tok_hbm.at[idxb[b]], bufs[b].at[pl.ds(_N0, _N1)], gsems[b]).start()

  def wait_gather(b):
    pltpu.make_async_copy(
        tok_hbm.at[idxa[b]], bufs[b].at[pl.ds(0, _N0)], gsems[b]).wait()
    pltpu.make_async_copy(
        tok_hbm.at[idxb[b]], bufs[b].at[pl.ds(_N0, _N1)], gsems[b]).wait()

  def start_scatter(g, b):
    base = (row0 + g) * _L
    pltpu.make_async_copy(
        bufs[b], out_hbm.at[pl.ds(base, _L)], ssems[b]).start()

  def wait_scatter(b):
    pltpu.make_async_copy(
        bufs[b], out_hbm.at[pl.ds(0, _L)], ssems[b]).wait()

  def add_pos(b):
    buf = bufs[b]
    def add_row(r, carry):
      for c in range(_HID // 16):
        sl = pl.ds(c * 16, 16)
        buf[r, sl] = buf[r, sl] + pos_v[r, sl]
      return carry
    lax.fori_loop(0, _L, add_row, 0, unroll=2)

  for j in range(_NBUF - 1):
    start_fetch(j, j)

  def outer(i, carry):
    for b in range(_NBUF):
      g = i * _NBUF + b
      r = g + (_NBUF - 1)
      @pl.when(jnp.logical_and(r < _ROWS_PER_W, g >= 1))
      def _():
        wait_scatter((b + _NBUF - 1) % _NBUF)

      @pl.when(r < _ROWS_PER_W)
      def _():
        start_fetch(r, (b + _NBUF - 1) % _NBUF)

      wait_gather(b)
      add_pos(b)
      start_scatter(g, b)
    return carry

  lax.fori_loop(0, _ROWS_PER_W // _NBUF, outer, 0)

  for b in range(_NBUF):
    wait_scatter(b)


@jax.jit
def _tpe_call(x_flat, tok_table, pos_table):
  mesh = plsc.VectorSubcoreMesh(core_axis_name="c", subcore_axis_name="s")
  kern = functools.partial(
      pl.kernel,
      mesh=mesh,
      compiler_params=pltpu.CompilerParams(use_tc_tiling_on_sc=False),
      out_type=jax.ShapeDtypeStruct((_B * _L, _HID), jnp.float32),
      scratch_types=(
          [pltpu.VMEM((_L, _HID), jnp.float32) for _ in range(_NBUF)]
          + [pltpu.VMEM((_N0,), jnp.int32) for _ in range(_NBUF)]
          + [pltpu.VMEM((_N1,), jnp.int32) for _ in range(_NBUF)]
          + [pltpu.VMEM((_L, _HID), jnp.float32)]
          + [pltpu.SemaphoreType.DMA] * (2 * _NBUF)
      ),
  )(_tpe_body)
  return kern(x_flat, tok_table, pos_table)


def kernel(x, tok_table, pos_table):
  x_flat = jnp.reshape(x.astype(jnp.int32), (_B * _L,))
  out = _tpe_call(x_flat, tok_table, pos_table)
  return jnp.reshape(out, (_B, _L, _HID))


# speedup vs baseline: 1.4102x; 1.4102x over previous
import functools

import jax
import jax.numpy as jnp
from jax import lax
from jax.experimental import pallas as pl
from jax.experimental.pallas import tpu as pltpu
from jax.experimental.pallas import tpu_sc as plsc

_HID = 64
_L = 200
_B = 4096
_NW = 32
_ROWS_PER_W = _B // _NW
_CROWS = 2
_CTOK = _CROWS * _L
_NCHUNK = _ROWS_PER_W // _CROWS
_NBUF = 3
_SPLITS = ((0, 104), (104, 96), (200, 104), (304, 96))


def _tpe_body(x_hbm, tok_hbm, pos_hbm, out_hbm, *scratch):
  bufs = scratch[0:_NBUF]
  idxs = scratch[_NBUF:2 * _NBUF]
  pos_v = scratch[2 * _NBUF]
  isems = scratch[2 * _NBUF + 1:2 * _NBUF + 1 + _NBUF]
  gsems = scratch[2 * _NBUF + 1 + _NBUF:2 * _NBUF + 1 + 2 * _NBUF]
  ssems = scratch[2 * _NBUF + 1 + 2 * _NBUF:]

  wid = lax.axis_index("s") * 2 + lax.axis_index("c")
  tok0 = wid * _ROWS_PER_W * _L

  pltpu.sync_copy(pos_hbm.at[pl.ds(0, _L)], pos_v)

  def start_idx(c, s):
    base = tok0 + c * _CTOK
    pltpu.make_async_copy(
        x_hbm.at[pl.ds(base, _CTOK)], idxs[s], isems[s]).start()

  def wait_idx(s):
    pltpu.make_async_copy(
        x_hbm.at[pl.ds(0, _CTOK)], idxs[s], isems[s]).wait()

  def start_gather(s):
    for (off, n) in _SPLITS:
      pltpu.make_async_copy(
          tok_hbm.at[idxs[s].at[pl.ds(off, n)]],
          bufs[s].at[pl.ds(off, n)],
          gsems[s],
      ).start()

  def wait_gather(s):
    pltpu.make_async_copy(
        tok_hbm.at[idxs[s]], bufs[s], gsems[s]).wait()

  def start_scatter(c, s):
    base = tok0 + c * _CTOK
    pltpu.make_async_copy(
        bufs[s], out_hbm.at[pl.ds(base, _CTOK)], ssems[s]).start()

  def wait_scatter(s):
    pltpu.make_async_copy(
        bufs[s], out_hbm.at[pl.ds(0, _CTOK)], ssems[s]).wait()

  def add_pos(s):
    buf = bufs[s]

    @plsc.parallel_loop(0, _L, 1, unroll=4)
    def _(r):
      for c in range(_HID // 16):
        sl = pl.ds(c * 16, 16)
        p = pos_v[r, sl]
        buf[r, sl] = buf[r, sl] + p
        buf[r + _L, sl] = buf[r + _L, sl] + p

  start_idx(0, 0)
  start_idx(1, 1)
  wait_idx(0)
  start_gather(0)

  def step(c, carry):
    for s in range(_NBUF):
      ci = c * _NBUF + s

      s1 = (s + 1) % _NBUF
      @pl.when(ci + 1 < _NCHUNK)
      def _():
        @pl.when(ci >= 2)
        def _():
          wait_scatter(s1)
        wait_idx(s1)
        start_gather(s1)

      @pl.when(ci + 2 < _NCHUNK)
      def _():
        start_idx(ci + 2, (s + 2) % _NBUF)

      wait_gather(s)
      add_pos(s)
      start_scatter(ci, s)
    return carry

  lax.fori_loop(0, _NCHUNK // _NBUF, step, 0)
  ci = _NCHUNK - 1
  s = ci % _NBUF
  wait_gather(s)
  add_pos(s)
  start_scatter(ci, s)

  for s in range(_NBUF):
    wait_scatter(s)


@jax.jit
def _tpe_call(x_flat, tok_table, pos_table):
  mesh = plsc.VectorSubcoreMesh(core_axis_name="c", subcore_axis_name="s")
  kern = functools.partial(
      pl.kernel,
      mesh=mesh,
      compiler_params=pltpu.CompilerParams(use_tc_tiling_on_sc=False),
      out_type=jax.ShapeDtypeStruct((_B * _L, _HID), jnp.float32),
      scratch_types=(
          [pltpu.VMEM((_CTOK, _HID), jnp.float32) for _ in range(_NBUF)]
          + [pltpu.VMEM((_CTOK,), jnp.int32) for _ in range(_NBUF)]
          + [pltpu.VMEM((_L, _HID), jnp.float32)]
          + [pltpu.SemaphoreType.DMA] * (3 * _NBUF)
      ),
  )(_tpe_body)
  return kern(x_flat, tok_table, pos_table)


def kernel(x, tok_table, pos_table):
  x_flat = jnp.reshape(x.astype(jnp.int32), (_B * _L,))
  out = _tpe_call(x_flat, tok_table, pos_table)
  return jnp.reshape(out, (_B, _L, _HID))
